# baseline (speedup 1.0000x reference)
import jax
import jax.numpy as jnp
from jax import lax
from jax.experimental import pallas as pl
from jax.experimental.pallas import tpu as pltpu

_B = 128
_V = 100000
_EPS = 1e-10
_BLK = 8192
_NB = (_V + _BLK - 1) // _BLK
_W = 512
_CPB = _BLK // _W
_NCH = _NB * _CPB
_J = 8
_RH = 64
_RG = 8
_NG = _B // _RG
_BIG = 2147483647

_KS0 = 0
_KS1 = 42
_KS2 = 0x1BD11BDA ^ 42
_ROT1 = (13, 15, 26, 6)
_ROT2 = (17, 29, 16, 24)


def _rotl(x, r):
    return lax.shift_left(x, jnp.int32(r)) | lax.shift_right_logical(
        x, jnp.int32(32 - r)
    )


def _threefry_bits(n):
    x0 = jnp.zeros_like(n)
    x1 = n + jnp.int32(_KS1)

    def rounds(x0, x1, rots):
        for r in rots:
            x0 = x0 + x1
            x1 = _rotl(x1, r)
            x1 = x1 ^ x0
        return x0, x1

    x0, x1 = rounds(x0, x1, _ROT1)
    x0 = x0 + jnp.int32(_KS1)
    x1 = x1 + jnp.int32(_KS2 + 1)
    x0, x1 = rounds(x0, x1, _ROT2)
    x0 = x0 + jnp.int32(_KS2)
    x1 = x1 + jnp.int32(_KS0 + 2)
    x0, x1 = rounds(x0, x1, _ROT1)
    x0 = x0 + jnp.int32(_KS0)
    x1 = x1 + jnp.int32(_KS1 + 3)
    x0, x1 = rounds(x0, x1, _ROT2)
    x0 = x0 + jnp.int32(_KS1)
    x1 = x1 + jnp.int32(_KS2 + 4)
    x0, x1 = rounds(x0, x1, _ROT1)
    x0 = x0 + jnp.int32(_KS2)
    x1 = x1 + jnp.int32(_KS0 + 5)
    return x0 ^ x1


def _bits_to_exp(bits):
    f = lax.bitcast_convert_type(
        lax.shift_right_logical(bits, jnp.int32(9)) | jnp.int32(0x3F800000),
        jnp.float32,
    )
    return -jnp.log1p(1.0 - f)


_TABLES = None


def _tables():
    global _TABLES
    if _TABLES is None:
        e = jax.random.exponential(jax.random.key(42), (_B, _V), jnp.float32)
        e = jnp.pad(e, ((0, 0), (0, _NCH * _W - _V)), constant_values=jnp.inf)
        ec = e.reshape(_B, _NCH, _W)
        negtop, idx = jax.lax.top_k(-ec, _J + 1)
        etop = -negtop
        pos = idx + (jnp.arange(_NCH, dtype=jnp.int32) * _W)[None, :, None]
        e_seed = etop[:, :, :_J]
        pos_seed = pos[:, :, :_J].astype(jnp.int32)
        e_cert = etop[:, :, _J]
        _TABLES = (
            jax.device_put(e_seed.reshape(_B, _NCH * _J)),
            jax.device_put(pos_seed.reshape(_B, _NCH * _J)),
            jax.device_put(e_cert),
        )
    return _TABLES


def _sweep1_body(l_ref, pos_ref, max_ref, idx_ref, cmax_ref, seedl_ref,
                 m_scr, i_scr):
    v = pl.program_id(1)

    @pl.when(v == 0)
    def _():
        m_scr[...] = jnp.full_like(m_scr, -jnp.inf)
        i_scr[...] = jnp.zeros_like(i_scr)

    l = l_ref[...]
    col = lax.broadcasted_iota(jnp.int32, l.shape, 1) + v * _BLK
    lm = jnp.where(col < _V, l, -jnp.inf)
    bm = jnp.max(lm, axis=1, keepdims=True)
    bi = jnp.min(jnp.where(lm == bm, col, _BIG), axis=1, keepdims=True)
    upd = bm > m_scr[...]
    i_scr[...] = jnp.where(upd, bi, i_scr[...])
    m_scr[...] = jnp.where(upd, bm, m_scr[...])

    l3 = lm.reshape(_RH, _CPB, _W)
    col3 = col.reshape(_RH, _CPB, _W)
    cmax_ref[...] = jnp.max(l3, axis=2)[None]
    pos = pos_ref[...].reshape(_RH, _CPB, _J)
    seeds = []
    for j in range(_J):
        pj = pos[:, :, j:j + 1]
        sj = jnp.max(jnp.where(col3 == pj, l3, -jnp.inf), axis=2)
        seeds.append(sj[:, :, None])
    seedl_ref[...] = jnp.concatenate(seeds, axis=2).reshape(_RH, _CPB * _J)

    @pl.when(v == _NB - 1)
    def _():
        max_ref[...] = m_scr[...]
        idx_ref[...] = i_scr[...]


def _sweep2_body(flags_ref, l_ref, t_ref, m_ref, vmax_ref, vidx_ref,
                 m_scr, i_scr):
    r = pl.program_id(0)
    v = pl.program_id(1)

    @pl.when(v == 0)
    def _():
        m_scr[...] = jnp.full_like(m_scr, -jnp.inf)
        i_scr[...] = jnp.zeros_like(i_scr)

    t = t_ref[...]
    m = m_ref[...]
    for c in range(_CPB):
        @pl.when(flags_ref[r * _NCH + v * _CPB + c] != 0)
        def _():
            l = l_ref[:, c * _W:(c + 1) * _W]
            row = lax.broadcasted_iota(jnp.int32, l.shape, 0) + r * _RG
            col = (lax.broadcasted_iota(jnp.int32, l.shape, 1)
                   + v * _BLK + c * _W)
            e = _bits_to_exp(_threefry_bits(row * _V + col))
            val = jnp.exp(l / t - m) / (e + _EPS)
            val = jnp.where(col < _V, val, -1.0)
            bm = jnp.max(val, axis=1, keepdims=True)
            bi = jnp.min(jnp.where(val == bm, col, _BIG), axis=1,
                         keepdims=True)
            upd = (bm > m_scr[...]) | ((bm == m_scr[...]) & (bi < i_scr[...]))
            i_scr[...] = jnp.where(upd, bi, i_scr[...])
            m_scr[...] = jnp.where(upd, bm, m_scr[...])

    @pl.when(v == _NB - 1)
    def _():
        vmax_ref[...] = m_scr[...]
        vidx_ref[...] = i_scr[...]


def kernel(logits, temperatures):
    logits = logits.astype(jnp.float32)
    e_seed, pos_seed, e_cert = _tables()

    row_spec = pl.BlockSpec((_RH, 1), lambda r, v: (r, 0))
    blk_spec = pl.BlockSpec((_RH, _BLK), lambda r, v: (r, v))

    lmax, gidx, cmax, seedl = pl.pallas_call(
        _sweep1_body,
        grid=(_B // _RH, _NB),
        in_specs=[
            blk_spec,
            pl.BlockSpec((_RH, _CPB * _J), lambda r, v: (r, v)),
        ],
        out_specs=[
            row_spec,
            row_spec,
            pl.BlockSpec((1, _RH, _CPB), lambda r, v: (v, r, 0)),
            pl.BlockSpec((_RH, _CPB * _J), lambda r, v: (r, v)),
        ],
        out_shape=[
            jax.ShapeDtypeStruct((_B, 1), jnp.float32),
            jax.ShapeDtypeStruct((_B, 1), jnp.int32),
            jax.ShapeDtypeStruct((_NB, _B, _CPB), jnp.float32),
            jax.ShapeDtypeStruct((_B, _NCH * _J), jnp.float32),
        ],
        scratch_shapes=[
            pltpu.VMEM((_RH, 1), jnp.float32),
            pltpu.VMEM((_RH, 1), jnp.int32),
        ],
        compiler_params=pltpu.CompilerParams(
            dimension_semantics=("parallel", "arbitrary"),
        ),
    )(logits, pos_seed)
    cmax = jnp.transpose(cmax, (1, 0, 2)).reshape(_B, _NCH)

    t_col = temperatures[:, None]
    m_col = lmax / t_col

    sv = jnp.exp(seedl / t_col - m_col) / (e_seed + _EPS)
    e_g = _bits_to_exp(
        _threefry_bits(jnp.arange(_B, dtype=jnp.int32) * _V + gidx[:, 0])
    )[:, None]
    gval = 1.0 / (e_g + _EPS)
    allv = jnp.concatenate([sv, gval], axis=1)
    allp = jnp.concatenate(
        [jnp.broadcast_to(pos_seed, sv.shape), gidx], axis=1
    )
    best = jnp.max(allv, axis=1, keepdims=True)
    bidx = jnp.min(jnp.where(allv == best, allp, _BIG), axis=1, keepdims=True)

    ub = jnp.exp(cmax / t_col - m_col) / (e_cert + _EPS)
    flags = (ub * (1.0 + 1e-4) >= best).reshape(_NG, _RG, _NCH).any(axis=1)
    flags = flags.astype(jnp.int32).reshape(_NG * _NCH)

    row_spec2 = pl.BlockSpec((_RG, 1), lambda r, v, *_: (r, 0))
    vmax, vidx = pl.pallas_call(
        _sweep2_body,
        grid_spec=pltpu.PrefetchScalarGridSpec(
            num_scalar_prefetch=1,
            grid=(_NG, _NB),
            in_specs=[
                pl.BlockSpec((_RG, _BLK), lambda r, v, *_: (r, v)),
                row_spec2,
                row_spec2,
            ],
            out_specs=[row_spec2, row_spec2],
            scratch_shapes=[
                pltpu.VMEM((_RG, 1), jnp.float32),
                pltpu.VMEM((_RG, 1), jnp.int32),
            ],
        ),
        out_shape=[
            jax.ShapeDtypeStruct((_B, 1), jnp.float32),
            jax.ShapeDtypeStruct((_B, 1), jnp.int32),
        ],
        compiler_params=pltpu.CompilerParams(
            dimension_semantics=("arbitrary", "arbitrary"),
        ),
    )(flags, logits, t_col, m_col)

    race_wins = (vmax > best) | ((vmax == best) & (vidx < bidx))
    sample = jnp.where(race_wins, vidx, bidx)
    out = jnp.where(t_col == 0.0, gidx, sample)
    return out[:, 0].astype(jnp.int32)


# speedup vs baseline: 11.1550x; 11.1550x over previous
import jax
import jax.numpy as jnp
from jax import lax
from jax.experimental import pallas as pl
from jax.experimental.pallas import tpu as pltpu

_B = 128
_V = 100000
_EPS = 1e-10
_BLK = 8192
_NB = (_V + _BLK - 1) // _BLK
_WLAST = 1792
_RH = 64
_RR = 32
_BIG = 2147483647

_NOISE_CHUNKS = None


def _noise_chunks():
    global _NOISE_CHUNKS
    if _NOISE_CHUNKS is None:
        e = jax.random.exponential(jax.random.key(42), (_B, _V), jnp.float32)
        r = 1.0 / (e + _EPS)
        r = jnp.pad(r, ((0, 0), (0, _NB * _BLK - _V)))
        _NOISE_CHUNKS = tuple(
            jax.device_put(r[:, k * _BLK:(k + 1) * _BLK] if k < _NB - 1
                           else r[:, k * _BLK:k * _BLK + _WLAST])
            for k in range(_NB)
        )
    return _NOISE_CHUNKS


def _greedy_body(l_ref, max_ref, idx_ref, m_scr, i_scr):
    v = pl.program_id(1)

    @pl.when(v == 0)
    def _():
        m_scr[...] = jnp.full_like(m_scr, -jnp.inf)
        i_scr[...] = jnp.zeros_like(i_scr)

    l = l_ref[...]
    col = lax.broadcasted_iota(jnp.int32, l.shape, 1) + v * _BLK
    lm = jnp.where(col < _V, l, -jnp.inf)
    bm = jnp.max(lm, axis=1, keepdims=True)
    bi = jnp.min(jnp.where(lm == bm, col, _BIG), axis=1, keepdims=True)
    upd = bm > m_scr[...]
    i_scr[...] = jnp.where(upd, bi, i_scr[...])
    m_scr[...] = jnp.where(upd, bm, m_scr[...])

    @pl.when(v == _NB - 1)
    def _():
        max_ref[...] = m_scr[...]
        idx_ref[...] = i_scr[...]


def _race_body(l_ref, t_ref, m_ref, g_ref, *rest):
    nz_refs = rest[:_NB]
    out_ref = rest[_NB]
    m_scr, i_scr = rest[_NB + 1:]
    v = pl.program_id(1)

    @pl.when(v == 0)
    def _():
        m_scr[...] = jnp.full_like(m_scr, -jnp.inf)
        i_scr[...] = jnp.zeros_like(i_scr)

    t = t_ref[...]
    m = m_ref[...]
    for k in range(_NB):
        @pl.when(v == k)
        def _(k=k):
            l = l_ref[...] if k < _NB - 1 else l_ref[:, :_WLAST]
            col = lax.broadcasted_iota(jnp.int32, l.shape, 1) + k * _BLK
            val = jnp.exp(l / t - m) * nz_refs[k][...]
            val = jnp.where(col < _V, val, -1.0)
            bm = jnp.max(val, axis=1, keepdims=True)
            bi = jnp.min(jnp.where(val == bm, col, _BIG), axis=1,
                         keepdims=True)
            upd = bm > m_scr[...]
            i_scr[...] = jnp.where(upd, bi, i_scr[...])
            m_scr[...] = jnp.where(upd, bm, m_scr[...])

    @pl.when(v == _NB - 1)
    def _():
        out_ref[...] = jnp.where(t == 0.0, g_ref[...], i_scr[...])


def kernel(logits, temperatures):
    logits = logits.astype(jnp.float32)
    chunks = _noise_chunks()

    row_spec = pl.BlockSpec((_RH, 1), lambda r, v: (r, 0))
    blk_spec = pl.BlockSpec((_RH, _BLK), lambda r, v: (r, v))

    lmax, gidx = pl.pallas_call(
        _greedy_body,
        grid=(_B // _RH, _NB),
        in_specs=[blk_spec],
        out_specs=[row_spec, row_spec],
        out_shape=[
            jax.ShapeDtypeStruct((_B, 1), jnp.float32),
            jax.ShapeDtypeStruct((_B, 1), jnp.int32),
        ],
        scratch_shapes=[
            pltpu.VMEM((_RH, 1), jnp.float32),
            pltpu.VMEM((_RH, 1), jnp.int32),
        ],
        compiler_params=pltpu.CompilerParams(
            dimension_semantics=("parallel", "arbitrary"),
        ),
    )(logits)

    t_col = temperatures[:, None]
    m_col = lmax / t_col

    row_spec2 = pl.BlockSpec((_RR, 1), lambda r, v: (r, 0))
    nz_specs = [
        pl.BlockSpec((_RR, _BLK if k < _NB - 1 else _WLAST),
                     lambda r, v: (r, 0))
        for k in range(_NB)
    ]
    out = pl.pallas_call(
        _race_body,
        grid=(_B // _RR, _NB),
        in_specs=[
            pl.BlockSpec((_RR, _BLK), lambda r, v: (r, v)),
            row_spec2,
            row_spec2,
            row_spec2,
        ] + nz_specs,
        out_specs=row_spec2,
        out_shape=jax.ShapeDtypeStruct((_B, 1), jnp.int32),
        scratch_shapes=[
            pltpu.VMEM((_RR, 1), jnp.float32),
            pltpu.VMEM((_RR, 1), jnp.int32),
        ],
        compiler_params=pltpu.CompilerParams(
            dimension_semantics=("arbitrary", "arbitrary"),
        ),
    )(logits, t_col, m_col, gidx, *chunks)

    return out[:, 0]


# speedup vs baseline: 11.5442x; 1.0349x over previous
import jax
import jax.numpy as jnp
from jax import lax
from jax.experimental import pallas as pl
from jax.experimental.pallas import tpu as pltpu

_B = 128
_V = 100000
_EPS = 1e-10
_BLK = 8192
_NB = (_V + _BLK - 1) // _BLK
_RH = 64
_RG = 8
_SLAB = 512
_BIG = 2147483647

_KS0 = 0
_KS1 = 42
_KS2 = 0x1BD11BDA ^ 42
_ROT1 = (13, 15, 26, 6)
_ROT2 = (17, 29, 16, 24)


def _rotl(x, r):
    return lax.shift_left(x, jnp.int32(r)) | lax.shift_right_logical(
        x, jnp.int32(32 - r)
    )


def _threefry_bits(n):
    x0 = jnp.zeros_like(n)
    x1 = n + jnp.int32(_KS1)

    def rounds(x0, x1, rots):
        for r in rots:
            x0 = x0 + x1
            x1 = _rotl(x1, r)
            x1 = x1 ^ x0
        return x0, x1

    x0, x1 = rounds(x0, x1, _ROT1)
    x0 = x0 + jnp.int32(_KS1)
    x1 = x1 + jnp.int32(_KS2 + 1)
    x0, x1 = rounds(x0, x1, _ROT2)
    x0 = x0 + jnp.int32(_KS2)
    x1 = x1 + jnp.int32(_KS0 + 2)
    x0, x1 = rounds(x0, x1, _ROT1)
    x0 = x0 + jnp.int32(_KS0)
    x1 = x1 + jnp.int32(_KS1 + 3)
    x0, x1 = rounds(x0, x1, _ROT2)
    x0 = x0 + jnp.int32(_KS1)
    x1 = x1 + jnp.int32(_KS2 + 4)
    x0, x1 = rounds(x0, x1, _ROT1)
    x0 = x0 + jnp.int32(_KS2)
    x1 = x1 + jnp.int32(_KS0 + 5)
    return x0 ^ x1


def _greedy_body(l_ref, max_ref, idx_ref, m_scr, i_scr):
    v = pl.program_id(1)

    @pl.when(v == 0)
    def _():
        m_scr[...] = jnp.full_like(m_scr, -jnp.inf)
        i_scr[...] = jnp.zeros_like(i_scr)

    l = l_ref[...]
    col = lax.broadcasted_iota(jnp.int32, l.shape, 1) + v * _BLK
    lm = jnp.where(col < _V, l, -jnp.inf)
    bm = jnp.max(lm, axis=1, keepdims=True)
    bi = jnp.min(jnp.where(lm == bm, col, _BIG), axis=1, keepdims=True)
    upd = bm > m_scr[...]
    i_scr[...] = jnp.where(upd, bi, i_scr[...])
    m_scr[...] = jnp.where(upd, bm, m_scr[...])

    @pl.when(v == _NB - 1)
    def _():
        max_ref[...] = m_scr[...]
        idx_ref[...] = i_scr[...]


def _race_body(l_ref, t_ref, m_ref, g_ref, out_ref, m_scr, i_scr):
    r = pl.program_id(0)
    v = pl.program_id(1)

    @pl.when(v == 0)
    def _():
        m_scr[...] = jnp.full_like(m_scr, -jnp.inf)
        i_scr[...] = jnp.zeros_like(i_scr)

    t = t_ref[...]
    m = m_ref[...]
    for c in range(_BLK // _SLAB):
        l = l_ref[:, c * _SLAB:(c + 1) * _SLAB]
        row = lax.broadcasted_iota(jnp.int32, l.shape, 0) + r * _RG
        col = (lax.broadcasted_iota(jnp.int32, l.shape, 1)
               + v * _BLK + c * _SLAB)
        bits = _threefry_bits(row * _V + col)
        f = lax.bitcast_convert_type(
            lax.shift_right_logical(bits, jnp.int32(9)) | jnp.int32(0x3F800000),
            jnp.float32,
        )
        e = -jnp.log1p(1.0 - f)
        val = jnp.exp(l / t - m) / (e + _EPS)
        val = jnp.where(col < _V, val, -1.0)
        bm = jnp.max(val, axis=1, keepdims=True)
        bi = jnp.min(jnp.where(val == bm, col, _BIG), axis=1, keepdims=True)
        upd = bm > m_scr[...]
        i_scr[...] = jnp.where(upd, bi, i_scr[...])
        m_scr[...] = jnp.where(upd, bm, m_scr[...])

    @pl.when(v == _NB - 1)
    def _():
        out_ref[...] = jnp.where(t == 0.0, g_ref[...], i_scr[...])


def kernel(logits, temperatures):
    logits = logits.astype(jnp.float32)

    row_spec = pl.BlockSpec((_RH, 1), lambda r, v: (r, 0))
    blk_spec = pl.BlockSpec((_RH, _BLK), lambda r, v: (r, v))

    lmax, gidx = pl.pallas_call(
        _greedy_body,
        grid=(_B // _RH, _NB),
        in_specs=[blk_spec],
        out_specs=[row_spec, row_spec],
        out_shape=[
            jax.ShapeDtypeStruct((_B, 1), jnp.float32),
            jax.ShapeDtypeStruct((_B, 1), jnp.int32),
        ],
        scratch_shapes=[
            pltpu.VMEM((_RH, 1), jnp.float32),
            pltpu.VMEM((_RH, 1), jnp.int32),
        ],
        compiler_params=pltpu.CompilerParams(
            dimension_semantics=("parallel", "arbitrary"),
        ),
    )(logits)

    t_col = temperatures[:, None]
    m_col = lmax / t_col

    row_spec2 = pl.BlockSpec((_RG, 1), lambda r, v: (r, 0))
    out = pl.pallas_call(
        _race_body,
        grid=(_B // _RG, _NB),
        in_specs=[
            pl.BlockSpec((_RG, _BLK), lambda r, v: (r, v)),
            row_spec2,
            row_spec2,
            row_spec2,
        ],
        out_specs=row_spec2,
        out_shape=jax.ShapeDtypeStruct((_B, 1), jnp.int32),
        scratch_shapes=[
            pltpu.VMEM((_RG, 1), jnp.float32),
            pltpu.VMEM((_RG, 1), jnp.int32),
        ],
        compiler_params=pltpu.CompilerParams(
            dimension_semantics=("arbitrary", "arbitrary"),
        ),
    )(logits, t_col, m_col, gidx)

    return out[:, 0]


# speedup vs baseline: 13.1673x; 1.1406x over previous
import jax
import jax.numpy as jnp
from jax import lax
from jax.experimental import pallas as pl
from jax.experimental.pallas import tpu as pltpu

_B = 128
_V = 100000
_EPS = 1e-10
_BLK = 8192
_NB = (_V + _BLK - 1) // _BLK
_RH = 64
_RG = 32
_SLAB = 512
_BIG = 2147483647

_KS0 = 0
_KS1 = 42
_KS2 = 0x1BD11BDA ^ 42
_ROT1 = (13, 15, 26, 6)
_ROT2 = (17, 29, 16, 24)


def _rotl(x, r):
    return lax.shift_left(x, jnp.int32(r)) | lax.shift_right_logical(
        x, jnp.int32(32 - r)
    )


def _threefry_bits(n):
    x0 = jnp.zeros_like(n)
    x1 = n + jnp.int32(_KS1)

    def rounds(x0, x1, rots):
        for r in rots:
            x0 = x0 + x1
            x1 = _rotl(x1, r)
            x1 = x1 ^ x0
        return x0, x1

    x0, x1 = rounds(x0, x1, _ROT1)
    x0 = x0 + jnp.int32(_KS1)
    x1 = x1 + jnp.int32(_KS2 + 1)
    x0, x1 = rounds(x0, x1, _ROT2)
    x0 = x0 + jnp.int32(_KS2)
    x1 = x1 + jnp.int32(_KS0 + 2)
    x0, x1 = rounds(x0, x1, _ROT1)
    x0 = x0 + jnp.int32(_KS0)
    x1 = x1 + jnp.int32(_KS1 + 3)
    x0, x1 = rounds(x0, x1, _ROT2)
    x0 = x0 + jnp.int32(_KS1)
    x1 = x1 + jnp.int32(_KS2 + 4)
    x0, x1 = rounds(x0, x1, _ROT1)
    x0 = x0 + jnp.int32(_KS2)
    x1 = x1 + jnp.int32(_KS0 + 5)
    return x0 ^ x1


def _greedy_body(l_ref, max_ref, idx_ref, m_scr, i_scr):
    v = pl.program_id(1)

    @pl.when(v == 0)
    def _():
        m_scr[...] = jnp.full_like(m_scr, -jnp.inf)
        i_scr[...] = jnp.zeros_like(i_scr)

    l = l_ref[...]
    col = lax.broadcasted_iota(jnp.int32, l.shape, 1) + v * _BLK
    lm = jnp.where(col < _V, l, -jnp.inf)
    bm = jnp.max(lm, axis=1, keepdims=True)
    bi = jnp.min(jnp.where(lm == bm, col, _BIG), axis=1, keepdims=True)
    upd = bm > m_scr[...]
    i_scr[...] = jnp.where(upd, bi, i_scr[...])
    m_scr[...] = jnp.where(upd, bm, m_scr[...])

    @pl.when(v == _NB - 1)
    def _():
        max_ref[...] = m_scr[...]
        idx_ref[...] = i_scr[...]


def _race_body(l_ref, t_ref, m_ref, g_ref, out_ref, m_scr, i_scr):
    r = pl.program_id(0)
    v = pl.program_id(1)

    @pl.when(v == 0)
    def _():
        m_scr[...] = jnp.full_like(m_scr, -jnp.inf)
        i_scr[...] = jnp.zeros_like(i_scr)

    for s in range(_RG // 8):
        t = t_ref[s * 8:(s + 1) * 8]
        m = m_ref[s * 8:(s + 1) * 8]
        for c in range(_BLK // _SLAB):
            l = l_ref[s * 8:(s + 1) * 8, c * _SLAB:(c + 1) * _SLAB]
            row = (lax.broadcasted_iota(jnp.int32, l.shape, 0)
                   + r * _RG + s * 8)
            col = (lax.broadcasted_iota(jnp.int32, l.shape, 1)
                   + v * _BLK + c * _SLAB)
            bits = _threefry_bits(row * _V + col)
            f = lax.bitcast_convert_type(
                lax.shift_right_logical(bits, jnp.int32(9))
                | jnp.int32(0x3F800000),
                jnp.float32,
            )
            e = -jnp.log1p(1.0 - f)
            val = jnp.exp(l / t - m) / (e + _EPS)
            val = jnp.where(col < _V, val, -1.0)
            bm = jnp.max(val, axis=1, keepdims=True)
            bi = jnp.min(jnp.where(val == bm, col, _BIG), axis=1,
                         keepdims=True)
            upd = bm > m_scr[s * 8:(s + 1) * 8]
            i_scr[s * 8:(s + 1) * 8] = jnp.where(upd, bi,
                                                 i_scr[s * 8:(s + 1) * 8])
            m_scr[s * 8:(s + 1) * 8] = jnp.where(upd, bm,
                                                 m_scr[s * 8:(s + 1) * 8])

    @pl.when(v == _NB - 1)
    def _():
        out_ref[...] = jnp.where(t_ref[...] == 0.0, g_ref[...], i_scr[...])


def kernel(logits, temperatures):
    logits = logits.astype(jnp.float32)

    row_spec = pl.BlockSpec((_RH, 1), lambda r, v: (r, 0))
    blk_spec = pl.BlockSpec((_RH, _BLK), lambda r, v: (r, v))

    lmax, gidx = pl.pallas_call(
        _greedy_body,
        grid=(_B // _RH, _NB),
        in_specs=[blk_spec],
        out_specs=[row_spec, row_spec],
        out_shape=[
            jax.ShapeDtypeStruct((_B, 1), jnp.float32),
            jax.ShapeDtypeStruct((_B, 1), jnp.int32),
        ],
        scratch_shapes=[
            pltpu.VMEM((_RH, 1), jnp.float32),
            pltpu.VMEM((_RH, 1), jnp.int32),
        ],
        compiler_params=pltpu.CompilerParams(
            dimension_semantics=("parallel", "arbitrary"),
        ),
    )(logits)

    t_col = temperatures[:, None]
    m_col = lmax / t_col

    row_spec2 = pl.BlockSpec((_RG, 1), lambda r, v: (r, 0))
    out = pl.pallas_call(
        _race_body,
        grid=(_B // _RG, _NB),
        in_specs=[
            pl.BlockSpec((_RG, _BLK), lambda r, v: (r, v)),
            row_spec2,
            row_spec2,
            row_spec2,
        ],
        out_specs=row_spec2,
        out_shape=jax.ShapeDtypeStruct((_B, 1), jnp.int32),
        scratch_shapes=[
            pltpu.VMEM((_RG, 1), jnp.float32),
            pltpu.VMEM((_RG, 1), jnp.int32),
        ],
        compiler_params=pltpu.CompilerParams(
            dimension_semantics=("arbitrary", "arbitrary"),
        ),
    )(logits, t_col, m_col, gidx)

    return out[:, 0]
